# baseline (speedup 1.0000x reference)
import functools

import jax
import jax.numpy as jnp
from jax import lax
from jax.experimental import pallas as pl
from jax.experimental.pallas import tpu as pltpu
from jax.experimental.pallas import tpu_sc as plsc

BATCH = 16384
NFIELD = 2
EMBED = 64
FLAT = BATCH * NFIELD
FIELD_OFFSET = 1000000
BN_EPS = 1e-5

_info = plsc.get_sparse_core_info()
_NC, _NS = _info.num_cores, _info.num_subcores
_NW = _NC * _NS
_BPW = FLAT // _NW
_CHUNK = 128
_NCHUNK = _BPW // _CHUNK


def _gather_body(emb_hbm, idx_hbm, out_hbm, idx_v, rows_v, sem):
    wid = lax.axis_index("s") * _NC + lax.axis_index("c")
    base = wid * _BPW
    pltpu.sync_copy(idx_hbm.at[pl.ds(wid * _NCHUNK, _NCHUNK)], idx_v)
    offs = (lax.iota(jnp.int32, 16) & 1) * FIELD_OFFSET
    for i in range(_NCHUNK):
        for j in range(_CHUNK // 16):
            sl = (i, pl.ds(j * 16, 16))
            idx_v[sl] = idx_v[sl] + offs
    copies = []
    for i in range(_NCHUNK):
        copies.append(
            pltpu.async_copy(
                emb_hbm.at[idx_v.at[i]],
                rows_v.at[pl.ds(i * _CHUNK, _CHUNK)],
                sem,
            )
        )
    for c in copies:
        c.wait()
    pltpu.sync_copy(rows_v, out_hbm.at[pl.ds(base, _BPW)])


_gather = functools.partial(
    pl.kernel,
    out_type=jax.ShapeDtypeStruct((FLAT, EMBED), jnp.float32),
    mesh=plsc.VectorSubcoreMesh(core_axis_name="c", subcore_axis_name="s"),
    scratch_types=[
        pltpu.VMEM((_NCHUNK, _CHUNK), jnp.int32),
        pltpu.VMEM((_BPW, EMBED), jnp.float32),
        pltpu.SemaphoreType.DMA,
    ],
    compiler_params=pltpu.CompilerParams(use_tc_tiling_on_sc=False),
)(_gather_body)


def _mlp_body(e_ref, w0, c0, w1, c1, w2, c2, wo, co, out_ref):
    h = e_ref[...]
    h = jnp.maximum(jnp.dot(h, w0[...], preferred_element_type=jnp.float32) + c0[...], 0.0)
    h = jnp.maximum(jnp.dot(h, w1[...], preferred_element_type=jnp.float32) + c1[...], 0.0)
    h = jnp.maximum(jnp.dot(h, w2[...], preferred_element_type=jnp.float32) + c2[...], 0.0)
    out_ref[...] = jnp.maximum(
        jnp.dot(h, wo[...], preferred_element_type=jnp.float32) + co[...], 0.0
    )


_MLP_BLK = 2048


def _mlp(e, w0, c0, w1, c1, w2, c2, wo, co):
    din = NFIELD * EMBED
    full = lambda shape: pl.BlockSpec(shape, lambda i: (0, 0))
    return pl.pallas_call(
        _mlp_body,
        grid=(BATCH // _MLP_BLK,),
        in_specs=[
            pl.BlockSpec((_MLP_BLK, din), lambda i: (i, 0)),
            full(w0.shape), full(c0.shape),
            full(w1.shape), full(c1.shape),
            full(w2.shape), full(c2.shape),
            full(wo.shape), full(co.shape),
        ],
        out_specs=pl.BlockSpec((_MLP_BLK, 1), lambda i: (i, 0)),
        out_shape=jax.ShapeDtypeStruct((BATCH, 1), jnp.float32),
    )(e, w0, c0, w1, c1, w2, c2, wo, co)


def kernel(x, emb, W0, b0, g0, be0, W1, b1, g1, be1, W2, b2, g2, be2, Wo, bo):
    idx2d = x.astype(jnp.int32).reshape(FLAT // _CHUNK, _CHUNK)
    gathered = _gather(emb, idx2d)
    e = gathered.reshape(BATCH, NFIELD * EMBED)

    inv = 1.0 / jnp.sqrt(jnp.float32(1.0 + BN_EPS))
    s0, s1, s2 = g0 * inv, g1 * inv, g2 * inv
    w0 = W0 * s0[None, :]
    c0 = (b0 * s0 + be0).reshape(1, -1)
    w1 = W1 * s1[None, :]
    c1 = (b1 * s1 + be1).reshape(1, -1)
    w2 = W2 * s2[None, :]
    c2 = (b2 * s2 + be2).reshape(1, -1)
    co = bo.reshape(1, 1)

    return _mlp(e, w0, c0, w1, c1, w2, c2, Wo, co)
